# baseline (speedup 1.0000x reference)
import functools

import jax
import jax.numpy as jnp
from jax.experimental import pallas as pl
from jax.experimental.pallas import tpu as pltpu

_K_OUT = 9
_RANKS = 17
_ROWS = 1024
_KPAD = 16
_LANES = 128


def _normalize_kernel(x_ref, xn_ref, xsq_ref):
    x = x_ref[0]
    norm = jnp.sqrt(jnp.sum(x * x, axis=0, keepdims=True))
    xn = x / jnp.maximum(norm, 1e-12)
    xn_ref[0] = xn
    xsq_ref[0] = jnp.sum(xn * xn, axis=0, keepdims=True)


def _knn_kernel(xn_ref, xsq_ref, out_ref, dist_ref, *, n_total):
    i = pl.program_id(1)

    nchunks = n_total // _LANES
    xr = xn_ref[0, :, pl.ds(i * _ROWS, _ROWS)]
    xsq_r = jnp.sum(xr * xr, axis=0)[:, None]

    ciota = jax.lax.broadcasted_iota(jnp.int32, (_ROWS, nchunks), 1)
    ciota3 = jax.lax.broadcasted_iota(jnp.int32, (nchunks, _ROWS, _LANES), 0)
    liota = jax.lax.broadcasted_iota(jnp.int32, (_ROWS, _LANES), 1)


    def initc(c, cm):
        xc = xn_ref[0, :, pl.ds(c * _LANES, _LANES)]
        s = jax.lax.dot_general(
            xr, xc,
            dimension_numbers=(((0,), (0,)), ((), ())),
            preferred_element_type=jnp.float32,
        )
        xsq_c = xsq_ref[0, :, pl.ds(c * _LANES, _LANES)]
        d = (xsq_r + (-2.0 * s)) + xsq_c
        dist_ref[c] = d
        return jnp.where(ciota == c, jnp.min(d, axis=1, keepdims=True), cm)

    cm0 = jax.lax.fori_loop(
        0, nchunks, initc, jnp.zeros((_ROWS, nchunks), jnp.float32))

    def body(k, carry):
        cm, lv, ll = carry
        v = jnp.min(cm, axis=1, keepdims=True)
        selc = jnp.min(jnp.where(cm == v, ciota, nchunks),
                       axis=1, keepdims=True)
        csel = ciota == selc
        lv_s = jnp.min(jnp.where(csel, lv, jnp.inf),
                       axis=1, keepdims=True)
        ll_s = jnp.min(jnp.where(csel, ll, _LANES),
                       axis=1, keepdims=True)
        e = jnp.min(jnp.where(ciota3 == selc, dist_ref[...], jnp.inf),
                    axis=0)

        valid = (e > lv_s) | ((e == lv_s) & (liota > ll_s))
        ev = jnp.where(valid, e, jnp.inf)
        pick_v = jnp.min(ev, axis=1, keepdims=True)
        pick_l = jnp.min(jnp.where(ev == pick_v, liota, _LANES),
                         axis=1, keepdims=True)
        ev2 = jnp.where((ev > pick_v) | ((ev == pick_v) & (liota > pick_l)),
                        ev, jnp.inf)
        newcm = jnp.min(ev2, axis=1, keepdims=True)

        idx = selc * _LANES + pick_l
        if k % 2 == 0:
            out_ref[0, :, k // 2:k // 2 + 1] = idx
        cm = jnp.where(csel, newcm, cm)
        lv = jnp.where(csel, pick_v, lv)
        ll = jnp.where(csel, pick_l, ll)
        return cm, lv, ll

    lv0 = jnp.full((_ROWS, nchunks), -jnp.inf, jnp.float32)
    ll0 = jnp.full((_ROWS, nchunks), -1, jnp.int32)
    carry = (cm0, lv0, ll0)
    for k in range(_RANKS):
        carry = body(k, carry)


def kernel(x):
    b, c, n, _ = x.shape
    x3 = x[..., 0]

    xn, xsq = pl.pallas_call(
        _normalize_kernel,
        grid=(b,),
        in_specs=[pl.BlockSpec((1, c, n), lambda bb: (bb, 0, 0))],
        out_specs=[
            pl.BlockSpec((1, c, n), lambda bb: (bb, 0, 0)),
            pl.BlockSpec((1, 1, n), lambda bb: (bb, 0, 0)),
        ],
        out_shape=[
            jax.ShapeDtypeStruct((b, c, n), jnp.float32),
            jax.ShapeDtypeStruct((b, 1, n), jnp.float32),
        ],
    )(x3)

    nn_pad = pl.pallas_call(
        functools.partial(_knn_kernel, n_total=n),
        grid=(b, n // _ROWS),
        in_specs=[
            pl.BlockSpec((1, c, n), lambda bb, ii: (bb, 0, 0)),
            pl.BlockSpec((1, 1, n), lambda bb, ii: (bb, 0, 0)),
        ],
        out_specs=pl.BlockSpec((1, _ROWS, _KPAD), lambda bb, ii: (bb, ii, 0)),
        out_shape=jax.ShapeDtypeStruct((b, n, _KPAD), jnp.int32),
        scratch_shapes=[
            pltpu.VMEM((n // _LANES, _ROWS, _LANES), jnp.float32),
        ],
    )(xn, xsq)

    nn_idx = nn_pad[:, :, :_K_OUT]
    center = jnp.broadcast_to(
        jnp.arange(n, dtype=jnp.int32)[None, :, None], (b, n, _K_OUT)
    )
    return jnp.stack((nn_idx, center), axis=0)


# speedup vs baseline: 1.0320x; 1.0320x over previous
import functools

import jax
import jax.numpy as jnp
from jax.experimental import pallas as pl
from jax.experimental.pallas import tpu as pltpu

_K_OUT = 9
_RANKS = 17
_ROWS = 1024
_KPAD = 16
_LANES = 128


def _normalize_kernel(x_ref, xn_ref, xsq_ref):
    x = x_ref[0]
    norm = jnp.sqrt(jnp.sum(x * x, axis=0, keepdims=True))
    xn = x / jnp.maximum(norm, 1e-12)
    xn_ref[0] = xn
    xsq_ref[0] = jnp.sum(xn * xn, axis=0, keepdims=True)


def _knn_kernel(xn_ref, xsq_ref, out_ref, dist_ref, *, n_total):
    i = pl.program_id(1)

    nchunks = n_total // _LANES
    xr = xn_ref[0, :, pl.ds(i * _ROWS, _ROWS)]
    xsq_r = jnp.sum(xr * xr, axis=0)[:, None]

    ciota = jax.lax.broadcasted_iota(jnp.int32, (_ROWS, nchunks), 1)
    ciota3 = jax.lax.broadcasted_iota(jnp.int32, (nchunks, _ROWS, _LANES), 0)
    liota = jax.lax.broadcasted_iota(jnp.int32, (_ROWS, _LANES), 1)
    kiota = jax.lax.broadcasted_iota(jnp.int32, (_ROWS, _KPAD), 1)


    def initc(c, cm):
        xc = xn_ref[0, :, pl.ds(c * _LANES, _LANES)]
        s = jax.lax.dot_general(
            xr, xc,
            dimension_numbers=(((0,), (0,)), ((), ())),
            preferred_element_type=jnp.float32,
        )
        xsq_c = xsq_ref[0, :, pl.ds(c * _LANES, _LANES)]
        d = (xsq_r + (-2.0 * s)) + xsq_c
        dist_ref[c] = d
        return jnp.where(ciota == c, jnp.min(d, axis=1, keepdims=True), cm)

    cm0 = jax.lax.fori_loop(
        0, nchunks, initc, jnp.zeros((_ROWS, nchunks), jnp.float32))

    def body(k, carry):
        acc, cm, lv, ll = carry
        v = jnp.min(cm, axis=1, keepdims=True)
        selc = jnp.min(jnp.where(cm == v, ciota, nchunks),
                       axis=1, keepdims=True)
        csel = ciota == selc
        lv_s = jnp.min(jnp.where(csel, lv, jnp.inf),
                       axis=1, keepdims=True)
        ll_s = jnp.min(jnp.where(csel, ll, _LANES),
                       axis=1, keepdims=True)
        e = jnp.min(jnp.where(ciota3 == selc, dist_ref[...], jnp.inf),
                    axis=0)

        valid = (e > lv_s) | ((e == lv_s) & (liota > ll_s))
        ev = jnp.where(valid, e, jnp.inf)
        pick_v = jnp.min(ev, axis=1, keepdims=True)
        pick_l = jnp.min(jnp.where(ev == pick_v, liota, _LANES),
                         axis=1, keepdims=True)
        ev2 = jnp.where((ev > pick_v) | ((ev == pick_v) & (liota > pick_l)),
                        ev, jnp.inf)
        newcm = jnp.min(ev2, axis=1, keepdims=True)

        idx = selc * _LANES + pick_l
        take = jnp.logical_and(k % 2 == 0, kiota == k // 2)
        acc = jnp.where(take, idx, acc)
        cm = jnp.where(csel, newcm, cm)
        lv = jnp.where(csel, pick_v, lv)
        ll = jnp.where(csel, pick_l, ll)
        return acc, cm, lv, ll

    acc0 = jnp.zeros((_ROWS, _KPAD), jnp.int32)
    lv0 = jnp.full((_ROWS, nchunks), -jnp.inf, jnp.float32)
    ll0 = jnp.full((_ROWS, nchunks), -1, jnp.int32)
    acc, _, _, _ = jax.lax.fori_loop(0, _RANKS, body, (acc0, cm0, lv0, ll0))
    out_ref[0] = acc


def kernel(x):
    b, c, n, _ = x.shape
    x3 = x[..., 0]

    xn, xsq = pl.pallas_call(
        _normalize_kernel,
        grid=(b,),
        in_specs=[pl.BlockSpec((1, c, n), lambda bb: (bb, 0, 0))],
        out_specs=[
            pl.BlockSpec((1, c, n), lambda bb: (bb, 0, 0)),
            pl.BlockSpec((1, 1, n), lambda bb: (bb, 0, 0)),
        ],
        out_shape=[
            jax.ShapeDtypeStruct((b, c, n), jnp.float32),
            jax.ShapeDtypeStruct((b, 1, n), jnp.float32),
        ],
    )(x3)

    nn_pad = pl.pallas_call(
        functools.partial(_knn_kernel, n_total=n),
        grid=(b, n // _ROWS),
        in_specs=[
            pl.BlockSpec((1, c, n), lambda bb, ii: (bb, 0, 0)),
            pl.BlockSpec((1, 1, n), lambda bb, ii: (bb, 0, 0)),
        ],
        out_specs=pl.BlockSpec((1, _ROWS, _KPAD), lambda bb, ii: (bb, ii, 0)),
        out_shape=jax.ShapeDtypeStruct((b, n, _KPAD), jnp.int32),
        scratch_shapes=[
            pltpu.VMEM((n // _LANES, _ROWS, _LANES), jnp.float32),
        ],
    )(xn, xsq)

    nn_idx = nn_pad[:, :, :_K_OUT]
    center = jnp.broadcast_to(
        jnp.arange(n, dtype=jnp.int32)[None, :, None], (b, n, _K_OUT)
    )
    return jnp.stack((nn_idx, center), axis=0)
